# Initial kernel scaffold; baseline (speedup 1.0000x reference)
import jax
import jax.numpy as jnp
from jax.experimental import pallas as pl


def kernel(input_data, centers, sigma, weights, Multicenter, Multiweight, Multisigma):
    raise NotImplementedError("write your pallas kernel here")


# speedup vs baseline: 2.2843x; 2.2843x over previous
import jax
import jax.numpy as jnp
from jax.experimental import pallas as pl
from jax.experimental.pallas import tpu as pltpu

_T_TILE = 512


def _fused(x_ref, c_ref, s_ref, w_ref, mc_ref, mw_ref, ms_ref,
           r_ref, rm_ref, pred_ref):
    x = x_ref[:, :]
    c = c_ref[:, :]
    s = s_ref[:, :]
    d = x - c
    r = jnp.exp(d * d * (-0.5 / (s * s)))
    r_ref[:, :] = r

    mc = mc_ref[:, :]
    ms = ms_ref[:, :]
    dm = x - mc
    rm = jnp.exp(dm * dm * (-0.5 / (ms * ms)))
    rm_ref[:, :] = rm

    pred = jnp.dot(w_ref[:, :], r, preferred_element_type=jnp.float32)
    pm = jnp.dot(mw_ref[:, :], rm, preferred_element_type=jnp.float32)
    pred_ref[:, :] = pred + pm


def kernel(input_data, centers, sigma, weights, Multicenter, Multiweight,
           Multisigma):
    T = input_data.shape[1]
    K = centers.shape[0]
    M = Multicenter.shape[0]
    F = weights.shape[0]
    tt = min(_T_TILE, T)
    grid = (T // tt,)

    full = lambda shape: pl.BlockSpec(shape, lambda i: (0, 0))
    return pl.pallas_call(
        _fused,
        grid=grid,
        in_specs=[
            pl.BlockSpec((1, tt), lambda i: (0, i)),
            full((K, 1)),
            full((K, 1)),
            full((F, K)),
            full((M, 1)),
            full((1, M)),
            full((M, 1)),
        ],
        out_specs=[
            pl.BlockSpec((K, tt), lambda i: (0, i)),
            pl.BlockSpec((M, tt), lambda i: (0, i)),
            pl.BlockSpec((F, tt), lambda i: (0, i)),
        ],
        out_shape=[
            jax.ShapeDtypeStruct((K, T), jnp.float32),
            jax.ShapeDtypeStruct((M, T), jnp.float32),
            jax.ShapeDtypeStruct((F, T), jnp.float32),
        ],
        compiler_params=pltpu.CompilerParams(
            dimension_semantics=("arbitrary",),
        ),
    )(input_data, centers, sigma, weights, Multicenter, Multiweight,
      Multisigma)
